# Initial kernel scaffold; baseline (speedup 1.0000x reference)
import jax
import jax.numpy as jnp
from jax.experimental import pallas as pl


def kernel(idx, a, b):
    raise NotImplementedError("write your pallas kernel here")


# speedup vs baseline: 8.2690x; 8.2690x over previous
import functools

import jax
import jax.numpy as jnp
from jax import lax
from jax.experimental import pallas as pl
from jax.experimental.pallas import tpu as pltpu
from jax.experimental.pallas import tpu_sc as plsc

NC, NS = 2, 16
NW = NC * NS
RANK = 32
DIM = 128


@functools.partial(jax.jit, static_argnames=("n", "chunk"))
def _sc_gather(idx_flat, table, *, n, chunk):
    per_w = n // NW
    n_chunks = per_w // chunk
    mesh = plsc.VectorSubcoreMesh(core_axis_name="c", subcore_axis_name="s")

    @functools.partial(
        pl.kernel,
        out_type=jax.ShapeDtypeStruct((n, RANK), jnp.float32),
        mesh=mesh,
        scratch_types=[
            pltpu.VMEM((chunk,), jnp.int32),
            pltpu.VMEM((chunk, RANK), jnp.float32),
            pltpu.SemaphoreType.DMA,
        ],
        compiler_params=pltpu.CompilerParams(use_tc_tiling_on_sc=False),
    )
    def gather_kernel(idx_hbm, table_hbm, out_hbm, idx_v, rows_v, sem):
        wid = lax.axis_index("s") * NC + lax.axis_index("c")
        base = wid * per_w

        def step(j, carry):
            off = base + j * chunk
            pltpu.sync_copy(idx_hbm.at[pl.ds(off, chunk)], idx_v)
            pltpu.async_copy(table_hbm.at[idx_v], rows_v, sem).wait()
            pltpu.sync_copy(rows_v, out_hbm.at[pl.ds(off, chunk)])
            return carry

        lax.fori_loop(0, n_chunks, step, 0)

    return gather_kernel(idx_flat, table)


def _mm_body(g_ref, b_ref, o_ref):
    o_ref[...] = jnp.dot(g_ref[...], b_ref[...],
                         preferred_element_type=jnp.float32)


@functools.partial(jax.jit, static_argnames=("n", "blk"))
def _tc_expand(gathered, b, *, n, blk):
    return pl.pallas_call(
        _mm_body,
        grid=(n // blk,),
        in_specs=[
            pl.BlockSpec((blk, RANK), lambda i: (i, 0)),
            pl.BlockSpec((RANK, DIM), lambda i: (0, 0)),
        ],
        out_specs=pl.BlockSpec((blk, DIM), lambda i: (i, 0)),
        out_shape=jax.ShapeDtypeStruct((n, DIM), jnp.float32),
    )(gathered, b)


def kernel(idx, a, b):
    bsz, feat = idx.shape
    n = bsz * feat
    idx_flat = idx.reshape(n)
    gathered = _sc_gather(idx_flat, a, n=n, chunk=1024)
    out = _tc_expand(gathered, b, n=n, blk=2048)
    return out.reshape(bsz, feat, DIM)


# speedup vs baseline: 26.4943x; 3.2041x over previous
import functools

import jax
import jax.numpy as jnp
from jax import lax
from jax.experimental import pallas as pl
from jax.experimental.pallas import tpu as pltpu
from jax.experimental.pallas import tpu_sc as plsc

NC, NS = 2, 16
NW = NC * NS
RANK = 32
DIM = 128


def _mm_body(aT_ref, b_ref, o_ref):
    o_ref[...] = lax.dot_general(
        aT_ref[...], b_ref[...], (((0,), (0,)), ((), ())),
        preferred_element_type=jnp.float32)


@functools.partial(jax.jit, static_argnames=("blk",))
def _expand_table(aT, b, *, blk):
    v = aT.shape[1]
    return pl.pallas_call(
        _mm_body,
        grid=(pl.cdiv(v, blk),),
        in_specs=[
            pl.BlockSpec((RANK, blk), lambda i: (0, i)),
            pl.BlockSpec((RANK, DIM), lambda i: (0, 0)),
        ],
        out_specs=pl.BlockSpec((blk, DIM), lambda i: (i, 0)),
        out_shape=jax.ShapeDtypeStruct((v, DIM), jnp.float32),
    )(aT, b)


@functools.partial(jax.jit, static_argnames=("n", "chunk"))
def _sc_gather(idx_flat, table2, *, n, chunk):
    per_w = n // NW
    n_chunks = per_w // chunk
    mesh = plsc.VectorSubcoreMesh(core_axis_name="c", subcore_axis_name="s")

    @functools.partial(
        pl.kernel,
        out_type=jax.ShapeDtypeStruct((n, DIM), jnp.float32),
        mesh=mesh,
        scratch_types=[
            pltpu.VMEM((chunk,), jnp.int32),
            pltpu.VMEM((chunk, DIM), jnp.float32),
            pltpu.SemaphoreType.DMA,
        ],
        compiler_params=pltpu.CompilerParams(use_tc_tiling_on_sc=False),
    )
    def gather_kernel(idx_hbm, table_hbm, out_hbm, idx_v, rows_v, sem):
        wid = lax.axis_index("s") * NC + lax.axis_index("c")
        base = wid * per_w

        def step(j, carry):
            off = base + j * chunk
            pltpu.sync_copy(idx_hbm.at[pl.ds(off, chunk)], idx_v)
            pltpu.async_copy(table_hbm.at[idx_v], rows_v, sem).wait()
            pltpu.sync_copy(rows_v, out_hbm.at[pl.ds(off, chunk)])
            return carry

        lax.fori_loop(0, n_chunks, step, 0)

    return gather_kernel(idx_flat, table2)


def kernel(idx, a, b):
    bsz, feat = idx.shape
    n = bsz * feat
    aT = jnp.transpose(a)
    idx_t = jnp.transpose(idx).reshape(n)
    table2 = _expand_table(aT, b, blk=8192)
    g = _sc_gather(idx_t, table2, n=n, chunk=832)
    return jnp.transpose(g.reshape(feat, bsz, DIM), (1, 0, 2))


# speedup vs baseline: 28.7649x; 1.0857x over previous
import functools

import jax
import jax.numpy as jnp
from jax import lax
from jax.experimental import pallas as pl
from jax.experimental.pallas import tpu as pltpu
from jax.experimental.pallas import tpu_sc as plsc

NC, NS = 2, 16
NW = NC * NS
RANK = 32
DIM = 128


def _mm_body(aT_ref, b_ref, o_ref):
    o_ref[...] = lax.dot_general(
        aT_ref[...], b_ref[...], (((0,), (0,)), ((), ())),
        preferred_element_type=jnp.float32)


@functools.partial(jax.jit, static_argnames=("blk",))
def _expand_table(aT, b, *, blk):
    v = aT.shape[1]
    return pl.pallas_call(
        _mm_body,
        grid=(pl.cdiv(v, blk),),
        in_specs=[
            pl.BlockSpec((RANK, blk), lambda i: (0, i)),
            pl.BlockSpec((RANK, DIM), lambda i: (0, 0)),
        ],
        out_specs=pl.BlockSpec((blk, DIM), lambda i: (i, 0)),
        out_shape=jax.ShapeDtypeStruct((v, DIM), jnp.float32),
    )(aT, b)


@functools.partial(jax.jit, static_argnames=("n", "chunk"))
def _sc_gather(idx_flat, table2, *, n, chunk):
    per_w = n // NW
    n_chunks = per_w // chunk
    n_groups = n_chunks // 2
    mesh = plsc.VectorSubcoreMesh(core_axis_name="c", subcore_axis_name="s")

    @functools.partial(
        pl.kernel,
        out_type=jax.ShapeDtypeStruct((n, DIM), jnp.float32),
        mesh=mesh,
        scratch_types=[
            pltpu.VMEM((per_w,), jnp.int32),
            pltpu.VMEM((chunk, DIM), jnp.float32),
            pltpu.VMEM((chunk, DIM), jnp.float32),
            pltpu.SemaphoreType.DMA,
            pltpu.SemaphoreType.DMA,
            pltpu.SemaphoreType.DMA,
            pltpu.SemaphoreType.DMA,
        ],
        compiler_params=pltpu.CompilerParams(use_tc_tiling_on_sc=False),
    )
    def gather_kernel(idx_hbm, table_hbm, out_hbm, idx_v, rows0, rows1,
                      gsem0, gsem1, wsem0, wsem1):
        wid = lax.axis_index("s") * NC + lax.axis_index("c")
        base = wid * per_w
        pltpu.sync_copy(idx_hbm.at[pl.ds(base, per_w)], idx_v)

        def fire(c, rows, gsem):
            pltpu.async_copy(
                table_hbm.at[idx_v.at[pl.ds(c * chunk, chunk)]], rows, gsem)

        def gwait(rows, gsem):
            pltpu.make_async_copy(
                table_hbm.at[pl.ds(0, chunk)], rows, gsem).wait()

        def wstart(c, rows, wsem):
            pltpu.async_copy(
                rows, out_hbm.at[pl.ds(base + c * chunk, chunk)], wsem)

        def wwait(rows, wsem):
            pltpu.make_async_copy(
                rows, out_hbm.at[pl.ds(base, chunk)], wsem).wait()

        fire(0, rows0, gsem0)
        fire(1, rows1, gsem1)

        def body(g, carry):
            c = 2 * g
            gwait(rows0, gsem0)
            wstart(c, rows0, wsem0)
            gwait(rows1, gsem1)
            wstart(c + 1, rows1, wsem1)
            wwait(rows0, wsem0)
            fire(c + 2, rows0, gsem0)
            wwait(rows1, wsem1)
            fire(c + 3, rows1, gsem1)
            return carry

        lax.fori_loop(0, n_groups - 1, body, 0)

        c_last = 2 * (n_groups - 1)
        gwait(rows0, gsem0)
        wstart(c_last, rows0, wsem0)
        gwait(rows1, gsem1)
        wstart(c_last + 1, rows1, wsem1)
        wwait(rows0, wsem0)
        wwait(rows1, wsem1)

    return gather_kernel(idx_flat, table2)


def kernel(idx, a, b):
    bsz, feat = idx.shape
    n = bsz * feat
    aT = jnp.transpose(a)
    idx_t = jnp.transpose(idx).reshape(n)
    table2 = _expand_table(aT, b, blk=16384)
    g = _sc_gather(idx_t, table2, n=n, chunk=416)
    return jnp.transpose(g.reshape(feat, bsz, DIM), (1, 0, 2))
